# baseline (speedup 1.0000x reference)
import functools

import jax
import jax.numpy as jnp
from jax import lax
from jax.experimental import pallas as pl
from jax.experimental.pallas import tpu as pltpu
from jax.experimental.pallas import tpu_sc as plsc

NC = 2
NS = 16
NW = NC * NS
B = 16384
D = 64
CHUNK = 128
B_PER_W = B // NW
N_CHUNKS = B_PER_W // CHUNK
ID_ROWS_PER_W = B_PER_W // CHUNK


def _sc_gather_body(uid_hbm, iid_hbm, ut_hbm, it_hbm, u_out, i_out,
                    uidx, iidx, urows, irows, sem_u, sem_i):
  c = lax.axis_index("c")
  s = lax.axis_index("s")
  wid = s * NC + c
  rbase = wid * ID_ROWS_PER_W
  base = wid * B_PER_W
  pltpu.sync_copy(uid_hbm.at[pl.ds(rbase, ID_ROWS_PER_W)], uidx)
  pltpu.sync_copy(iid_hbm.at[pl.ds(rbase, ID_ROWS_PER_W)], iidx)
  copies = []
  for j in range(N_CHUNKS):
    dst = pl.ds(j * CHUNK, CHUNK)
    copies.append(pltpu.async_copy(ut_hbm.at[uidx.at[j]], urows.at[dst], sem_u))
    copies.append(pltpu.async_copy(it_hbm.at[iidx.at[j]], irows.at[dst], sem_i))
  for cp in copies:
    cp.wait()
  pltpu.sync_copy(urows, u_out.at[pl.ds(base, B_PER_W)])
  pltpu.sync_copy(irows, i_out.at[pl.ds(base, B_PER_W)])


_sc_gather = pl.kernel(
    _sc_gather_body,
    out_type=(
        jax.ShapeDtypeStruct((B, D), jnp.float32),
        jax.ShapeDtypeStruct((B, D), jnp.float32),
    ),
    mesh=plsc.VectorSubcoreMesh(core_axis_name="c", subcore_axis_name="s"),
    compiler_params=pltpu.CompilerParams(use_tc_tiling_on_sc=False),
    scratch_types=[
        pltpu.VMEM((ID_ROWS_PER_W, CHUNK), jnp.int32),
        pltpu.VMEM((ID_ROWS_PER_W, CHUNK), jnp.int32),
        pltpu.VMEM((B_PER_W, D), jnp.float32),
        pltpu.VMEM((B_PER_W, D), jnp.float32),
        pltpu.SemaphoreType.DMA,
        pltpu.SemaphoreType.DMA,
    ],
)


BLK = 2048


def _mlp_body(u_ref, i_ref, w1u_ref, w1i_ref, b1_ref, w2_ref, b2_ref,
              w3_ref, b3_ref, o_ref):
  h = (jnp.dot(u_ref[...], w1u_ref[...], preferred_element_type=jnp.float32)
       + jnp.dot(i_ref[...], w1i_ref[...], preferred_element_type=jnp.float32)
       + b1_ref[...])
  h = jnp.maximum(h, 0.0)
  h = jnp.dot(h, w2_ref[...], preferred_element_type=jnp.float32) + b2_ref[...]
  h = jnp.maximum(h, 0.0)
  o_ref[...] = (jnp.dot(h, w3_ref[...], preferred_element_type=jnp.float32)
                + b3_ref[...])


_mlp = pl.pallas_call(
    _mlp_body,
    grid=(B // BLK,),
    in_specs=[
        pl.BlockSpec((BLK, D), lambda b: (b, 0)),
        pl.BlockSpec((BLK, D), lambda b: (b, 0)),
        pl.BlockSpec((D, 128), lambda b: (0, 0)),
        pl.BlockSpec((D, 128), lambda b: (0, 0)),
        pl.BlockSpec((1, 128), lambda b: (0, 0)),
        pl.BlockSpec((128, 64), lambda b: (0, 0)),
        pl.BlockSpec((1, 64), lambda b: (0, 0)),
        pl.BlockSpec((D, 1), lambda b: (0, 0)),
        pl.BlockSpec((1, 1), lambda b: (0, 0)),
    ],
    out_specs=pl.BlockSpec((BLK, 1), lambda b: (b, 0)),
    out_shape=jax.ShapeDtypeStruct((B, 1), jnp.float32),
)


@jax.jit
def kernel(user_ids, item_ids, user_table, item_table, W1, b1, W2, b2, W3, b3):
  uid2 = user_ids.reshape(B // CHUNK, CHUNK)
  iid2 = item_ids.reshape(B // CHUNK, CHUNK)
  u_e, i_e = _sc_gather(uid2, iid2, user_table, item_table)
  w1u = W1[:, :D].T
  w1i = W1[:, D:].T
  out = _mlp(u_e, i_e, w1u, w1i, b1[None, :], W2.T, b2[None, :],
             W3.T, b3[None, :])
  return out[:, 0]
